# baseline (speedup 1.0000x reference)
import functools

import jax
import jax.numpy as jnp
from jax import lax
from jax.experimental import pallas as pl
from jax.experimental.pallas import tpu as pltpu
from jax.experimental.pallas import tpu_sc as plsc

N_PIX = 65536
D = 128
K_MEM = 100000
P_CLU = 2048
NUM_CLASSES = 21
TOP_K = 20

NB = 784
K_PAD = NB * 128
COL_BLK = 1024
NEG = -3.4e38


def _tc_sims_body(acc_ref, mem_ref, sims_ref, m_ref, ph_ref):
    i = pl.program_id(0)

    @pl.when(i == 0)
    def _():
        proto = acc_ref[0] + acc_ref[1]
        pn = jnp.sqrt(jnp.sum(proto * proto, axis=1, keepdims=True))
        ph_ref[...] = (proto / (pn + 1e-12)).astype(jnp.bfloat16)

    mb = mem_ref[...]
    mn = jnp.sqrt(jnp.sum(mb * mb, axis=1, keepdims=True))
    mh = (mb / (mn + 1e-12)).astype(jnp.bfloat16)
    s = lax.dot_general(ph_ref[...], mh, (((1,), (1,)), ((), ())),
                        preferred_element_type=jnp.float32)

    @pl.when(i != K_MEM // COL_BLK)
    def _():
        sims_ref[...] = s.reshape(P_CLU, 1, COL_BLK // 128, 128)
        m_ref[0] = jnp.max(s.reshape(P_CLU, COL_BLK // 128, 128), axis=2)

    @pl.when(i == K_MEM // COL_BLK)
    def _():
        col = i * COL_BLK + lax.broadcasted_iota(jnp.int32, s.shape, 1)
        sm = jnp.where(col < K_MEM, s, NEG)
        sims_ref[...] = sm.reshape(P_CLU, 1, COL_BLK // 128, 128)
        m_ref[0] = jnp.max(sm.reshape(P_CLU, COL_BLK // 128, 128), axis=2)


def _tc_sims(acc, mem_pad):
    grid = (K_PAD // COL_BLK,)
    return pl.pallas_call(
        _tc_sims_body,
        grid=grid,
        in_specs=[
            pl.BlockSpec((2, P_CLU, D), lambda i: (0, 0, 0)),
            pl.BlockSpec((COL_BLK, D), lambda i: (i, 0)),
        ],
        out_specs=[
            pl.BlockSpec((P_CLU, 1, COL_BLK // 128, 128), lambda i: (0, i, 0, 0)),
            pl.BlockSpec((1, P_CLU, COL_BLK // 128), lambda i: (i, 0, 0)),
        ],
        out_shape=[
            jax.ShapeDtypeStruct((P_CLU, K_PAD // COL_BLK, COL_BLK // 128, 128),
                                 jnp.float32),
            jax.ShapeDtypeStruct((K_PAD // COL_BLK, P_CLU, COL_BLK // 128),
                                 jnp.float32),
        ],
        scratch_shapes=[pltpu.VMEM((P_CLU, D), jnp.bfloat16)],
    )(acc, mem_pad)


_NC = 2
_NW = 32
_RPW = P_CLU // _NW
_NG = _RPW // 16
_NQ = NB // 16


def _sc2_body(sims_hbm, m_hbm, lab_hbm, labout_hbm, majout_hbm,
              m_slab, mt, l1, blk, outb, majb, labs, lab16, pbk, pp, sem):
    wid = lax.axis_index("s") * _NC + lax.axis_index("c")
    r0 = wid * _RPW
    pltpu.sync_copy(m_hbm.at[pl.ds(r0 * NB, _RPW * NB)], m_slab)
    lanes = lax.iota(jnp.int32, 16)
    ninf = jnp.full((16,), -jnp.inf, jnp.float32)
    zero16 = jnp.zeros((16,), jnp.int32)

    def group_body(g, _):
        rbase = r0 + g * 16
        def mtb(k, c):
            v = plsc.load_gather(
                m_slab, [(g * 16 + lanes) * NB + k])
            mt[pl.ds(k * 16, 16)] = v
            return c
        lax.fori_loop(0, NB, mtb, 0)

        def l1b(q, c):
            acc = ninf
            for j in range(16):
                acc = jnp.maximum(acc, mt[pl.ds((q * 16 + j) * 16, 16)])
            l1[pl.ds(q * 16, 16)] = acc
            return c
        lax.fori_loop(0, _NQ, l1b, 0)

        def round_body(r, c):
            def scan_q(q, carry):
                bv, bq = carry
                v = l1[pl.ds(q * 16, 16)]
                better = v > bv
                return (jnp.where(better, v, bv), jnp.where(better, q, bq))
            bv, bq = lax.fori_loop(0, _NQ, scan_q, (ninf, zero16))
            lv, lk = ninf, zero16
            for j in range(16):
                kidx = bq * 16 + j
                v = plsc.load_gather(mt, [kidx * 16 + lanes])
                better = v > lv
                lv = jnp.where(better, v, lv)
                lk = jnp.where(better, kidx, lk)
            handles = []
            for rr in range(16):
                off = (rbase + rr) * K_PAD + lk[rr] * 128
                handles.append(pltpu.async_copy(
                    sims_hbm.at[pl.ds(off, 128)],
                    blk.at[pl.ds(rr * 128, 128)], sem))
            for h in handles:
                h.wait()
            def prior_scan(rp, ws):
                mbk = pbk[pl.ds(rp * 16, 16)]
                ppos = pp[pl.ds(rp * 16, 16)]
                match = (mbk == lk) & (rp < r)
                bit = jnp.where(match,
                                jnp.left_shift(jnp.full((16,), 1, jnp.int32),
                                               ppos & 31), 0)
                wsel = ppos >> 5
                return tuple(w | jnp.where(wsel == i, bit, 0)
                             for i, w in enumerate(ws))
            ws = lax.fori_loop(0, TOP_K, prior_scan,
                               (zero16, zero16, zero16, zero16))
            m1, p1, m2 = ninf, zero16, ninf
            for w in range(4):
                def scan_j(j, carry, _w=w, _mask=ws[w]):
                    m1, p1, m2 = carry
                    x = plsc.load_gather(
                        blk, [lanes * 128 + (_w * 32 + j)])
                    excl = jnp.right_shift(_mask, j) & 1
                    x = jnp.where(excl == 1, -jnp.inf, x)
                    upd = x > m1
                    m2 = jnp.where(upd, m1, jnp.maximum(m2, x))
                    p1 = jnp.where(upd, _w * 32 + j, p1)
                    m1 = jnp.where(upd, x, m1)
                    return (m1, p1, m2)
                m1, p1, m2 = lax.fori_loop(0, 32, scan_j, (m1, p1, m2))
            pbk[pl.ds(r * 16, 16)] = lk
            pp[pl.ds(r * 16, 16)] = p1
            col = lk * 128 + p1
            pltpu.async_copy(lab_hbm.at[col], lab16, sem).wait()
            lv16 = lab16[...]
            labs[pl.ds(r * 16, 16)] = lv16
            plsc.store_scatter(outb, [lanes * TOP_K + r], lv16)
            plsc.store_scatter(mt, [lk * 16 + lanes], m2)
            acc = ninf
            for j in range(16):
                acc = jnp.maximum(
                    acc, plsc.load_gather(mt, [(bq * 16 + j) * 16 + lanes]))
            plsc.store_scatter(l1, [bq * 16 + lanes], acc)
            return c
        lax.fori_loop(0, TOP_K, round_body, 0)

        def majj(j, cnts):
            lvv = labs[pl.ds(j * 16, 16)]
            return tuple(cn + (lvv == cc).astype(jnp.int32)
                         for cc, cn in enumerate(cnts))
        cnts = lax.fori_loop(0, TOP_K, majj,
                             tuple(zero16 for _ in range(NUM_CLASSES)))
        bestc, bestn = zero16, cnts[0]
        for cc in range(1, NUM_CLASSES):
            better = cnts[cc] > bestn
            bestn = jnp.where(better, cnts[cc], bestn)
            bestc = jnp.where(better, jnp.full((16,), cc, jnp.int32), bestc)
        majb[...] = bestc
        pltpu.sync_copy(majb, majout_hbm.at[pl.ds(rbase, 16)])
        pltpu.sync_copy(outb, labout_hbm.at[pl.ds(rbase * TOP_K, 16 * TOP_K)])
        return _
    lax.fori_loop(0, _NG, group_body, 0)


def _sc_topk(sims2, m_t, labels):
    mesh = plsc.VectorSubcoreMesh(core_axis_name="c", subcore_axis_name="s")
    f = functools.partial(
        pl.kernel,
        out_type=[jax.ShapeDtypeStruct((P_CLU * TOP_K,), jnp.int32),
                  jax.ShapeDtypeStruct((P_CLU,), jnp.int32)],
        mesh=mesh,
        compiler_params=pltpu.CompilerParams(use_tc_tiling_on_sc=False,
                                             needs_layout_passes=False),
        scratch_types=[
            pltpu.VMEM((_RPW * NB,), jnp.float32),
            pltpu.VMEM((NB * 16,), jnp.float32),
            pltpu.VMEM((_NQ * 16,), jnp.float32),
            pltpu.VMEM((16 * 128,), jnp.float32),
            pltpu.VMEM((16 * TOP_K,), jnp.int32),
            pltpu.VMEM((16,), jnp.int32),
            pltpu.VMEM((TOP_K * 16,), jnp.int32),
            pltpu.VMEM((16,), jnp.int32),
            pltpu.VMEM((TOP_K * 16,), jnp.int32),
            pltpu.VMEM((TOP_K * 16,), jnp.int32),
            pltpu.SemaphoreType.DMA,
        ])(_sc2_body)
    lab_flat, maj = f(sims2, m_t.reshape(-1), labels)
    return lab_flat, maj


_PPW = N_PIX // _NW
_CHUNK = 128


def _sc1_body(emb_hbm, ci_hbm, zeros_hbm, out_hbm, rows_v, idxv, acc_sh, sem):
    c = lax.axis_index("c")
    s = lax.axis_index("s")
    wid = s * _NC + c
    pltpu.sync_copy(zeros_hbm, acc_sh.at[pl.ds(s * 128, 128)])
    plsc.subcore_barrier()
    def win(t, _):
        base = wid * _PPW + t * _CHUNK
        pltpu.sync_copy(ci_hbm.at[pl.ds(base, _CHUNK)], idxv)
        pltpu.sync_copy(emb_hbm.at[pl.ds(base, _CHUNK)], rows_v)
        pltpu.sync_copy(rows_v, acc_sh.at[idxv], add=True)
        return _
    lax.fori_loop(0, _PPW // _CHUNK, win, 0)
    plsc.subcore_barrier()
    pltpu.sync_copy(acc_sh.at[pl.ds(s * 128, 128)],
                    out_hbm.at[pl.ds(c * P_CLU + s * 128, 128)])


def _sc_scatter_add(emb, ci, zeros):
    mesh = plsc.VectorSubcoreMesh(core_axis_name="c", subcore_axis_name="s")
    f = functools.partial(
        pl.kernel,
        out_type=jax.ShapeDtypeStruct((_NC * P_CLU, D), jnp.float32),
        mesh=mesh,
        compiler_params=pltpu.CompilerParams(use_tc_tiling_on_sc=False,
                                             needs_layout_passes=False),
        scratch_types=[
            pltpu.VMEM((_CHUNK, D), jnp.float32),
            pltpu.VMEM((_CHUNK,), jnp.int32),
            pltpu.VMEM_SHARED((P_CLU, D), jnp.float32),
            pltpu.SemaphoreType.DMA,
        ])(_sc1_body)
    return f(emb, ci, zeros)


def _sc3_body(maj_hbm, lab_hbm, ci_hbm, pred_hbm, topk_hbm,
              majv, labv, cidx, predb, outb, sem):
    wid = lax.axis_index("s") * _NC + lax.axis_index("c")
    p0 = wid * _PPW
    pltpu.sync_copy(maj_hbm, majv)
    pltpu.sync_copy(lab_hbm, labv)
    lanes = lax.iota(jnp.int32, 16)

    def chunk_body(t, _):
        base = p0 + t * _CHUNK
        pltpu.sync_copy(ci_hbm.at[pl.ds(base, _CHUNK)], cidx)
        def sub_body(u, __):
            c16 = cidx[pl.ds(u * 16, 16)]
            pred16 = plsc.load_gather(majv, [c16])
            predb[pl.ds(u * 16, 16)] = pred16
            lpix = u * 16 + lanes
            for j in range(TOP_K):
                l16 = plsc.load_gather(labv, [c16 * TOP_K + j])
                plsc.store_scatter(outb, [lpix * TOP_K + j], l16)
            return __
        lax.fori_loop(0, _CHUNK // 16, sub_body, 0)
        pltpu.sync_copy(predb, pred_hbm.at[pl.ds(base, _CHUNK)])
        pltpu.sync_copy(outb, topk_hbm.at[pl.ds(base * TOP_K,
                                                _CHUNK * TOP_K)])
        return _
    lax.fori_loop(0, _PPW // _CHUNK, chunk_body, 0)


def _sc_broadcast(maj, lab_flat, ci):
    mesh = plsc.VectorSubcoreMesh(core_axis_name="c", subcore_axis_name="s")
    f = functools.partial(
        pl.kernel,
        out_type=[jax.ShapeDtypeStruct((N_PIX,), jnp.int32),
                  jax.ShapeDtypeStruct((N_PIX * TOP_K,), jnp.int32)],
        mesh=mesh,
        compiler_params=pltpu.CompilerParams(use_tc_tiling_on_sc=False,
                                             needs_layout_passes=False),
        scratch_types=[
            pltpu.VMEM((P_CLU,), jnp.int32),
            pltpu.VMEM((P_CLU * TOP_K,), jnp.int32),
            pltpu.VMEM((_CHUNK,), jnp.int32),
            pltpu.VMEM((_CHUNK,), jnp.int32),
            pltpu.VMEM((_CHUNK * TOP_K,), jnp.int32),
            pltpu.SemaphoreType.DMA,
        ])(_sc3_body)
    pred, topk_flat = f(maj, lab_flat, ci)
    return pred, topk_flat.reshape(N_PIX, TOP_K)


def kernel(cluster_embedding, cluster_index, memory_prototype,
           memory_prototype_label):
    ci = cluster_index.astype(jnp.int32)

    zeros = jnp.zeros((128, D), jnp.float32)
    acc = _sc_scatter_add(cluster_embedding, ci, zeros).reshape(2, P_CLU, D)

    mem_pad = jnp.pad(memory_prototype, ((0, K_PAD - K_MEM), (0, 0)))
    sims, blk_max = _tc_sims(acc, mem_pad)
    blk_max = jnp.transpose(blk_max, (1, 0, 2)).reshape(P_CLU, NB)

    sims2 = sims.reshape(P_CLU * K_PAD)
    lab_flat, maj = _sc_topk(sims2, blk_max,
                             memory_prototype_label.astype(jnp.int32))

    semantic_pred, semantic_topk = _sc_broadcast(maj, lab_flat, ci)
    return semantic_pred, semantic_topk


# speedup vs baseline: 1.2489x; 1.2489x over previous
import functools

import jax
import jax.numpy as jnp
from jax import lax
from jax.experimental import pallas as pl
from jax.experimental.pallas import tpu as pltpu
from jax.experimental.pallas import tpu_sc as plsc

N_PIX = 65536
D = 128
K_MEM = 100000
P_CLU = 2048
NUM_CLASSES = 21
TOP_K = 20

NB = 784
K_PAD = NB * 128
COL_BLK = 1024
NEG = -3.4e38


def _tc_sims_body(acc_ref, mem_ref, sims_ref, m_ref, ph_ref):
    i = pl.program_id(0)

    @pl.when(i == 0)
    def _():
        proto = acc_ref[0] + acc_ref[1]
        pn = jnp.sqrt(jnp.sum(proto * proto, axis=1, keepdims=True))
        ph_ref[...] = (proto / (pn + 1e-12)).astype(jnp.bfloat16)

    mb = mem_ref[...]
    mn = jnp.sqrt(jnp.sum(mb * mb, axis=1, keepdims=True))
    mh = (mb / (mn + 1e-12)).astype(jnp.bfloat16)
    s = lax.dot_general(ph_ref[...], mh, (((1,), (1,)), ((), ())),
                        preferred_element_type=jnp.float32)

    @pl.when(i != K_MEM // COL_BLK)
    def _():
        sims_ref[0] = s.reshape(P_CLU, COL_BLK // 128, 128)
        m_ref[0] = jnp.max(s.reshape(P_CLU, COL_BLK // 128, 128), axis=2)

    @pl.when(i == K_MEM // COL_BLK)
    def _():
        col = i * COL_BLK + lax.broadcasted_iota(jnp.int32, s.shape, 1)
        sm = jnp.where(col < K_MEM, s, NEG)
        sims_ref[0] = sm.reshape(P_CLU, COL_BLK // 128, 128)
        m_ref[0] = jnp.max(sm.reshape(P_CLU, COL_BLK // 128, 128), axis=2)


def _tc_sims(acc, mem_pad):
    grid = (K_PAD // COL_BLK,)
    return pl.pallas_call(
        _tc_sims_body,
        grid=grid,
        in_specs=[
            pl.BlockSpec((2, P_CLU, D), lambda i: (0, 0, 0)),
            pl.BlockSpec((COL_BLK, D), lambda i: (i, 0)),
        ],
        out_specs=[
            pl.BlockSpec((1, P_CLU, COL_BLK // 128, 128), lambda i: (i, 0, 0, 0)),
            pl.BlockSpec((1, P_CLU, COL_BLK // 128), lambda i: (i, 0, 0)),
        ],
        out_shape=[
            jax.ShapeDtypeStruct((K_PAD // COL_BLK, P_CLU, COL_BLK // 128, 128),
                                 jnp.float32),
            jax.ShapeDtypeStruct((K_PAD // COL_BLK, P_CLU, COL_BLK // 128),
                                 jnp.float32),
        ],
        scratch_shapes=[pltpu.VMEM((P_CLU, D), jnp.bfloat16)],
    )(acc, mem_pad)


_NC = 2
_NW = 32
_RPW = P_CLU // _NW
_NG = _RPW // 16
_NQ = NB // 16


def _sc2_body(sims_hbm, m_hbm, lab_hbm, labout_hbm, majout_hbm,
              m_slab, mt, l1, blk, outb, majb, labs, lab16, pbk, pp, sem):
    wid = lax.axis_index("s") * _NC + lax.axis_index("c")
    r0 = wid * _RPW
    pltpu.sync_copy(m_hbm.at[pl.ds(r0 * NB, _RPW * NB)], m_slab)
    lanes = lax.iota(jnp.int32, 16)
    ninf = jnp.full((16,), -jnp.inf, jnp.float32)
    zero16 = jnp.zeros((16,), jnp.int32)

    def group_body(g, _):
        rbase = r0 + g * 16
        def mtb(k, c):
            v = plsc.load_gather(
                m_slab, [(g * 16 + lanes) * NB + k])
            mt[pl.ds(k * 16, 16)] = v
            return c
        lax.fori_loop(0, NB, mtb, 0)

        def l1b(q, c):
            acc = ninf
            for j in range(16):
                acc = jnp.maximum(acc, mt[pl.ds((q * 16 + j) * 16, 16)])
            l1[pl.ds(q * 16, 16)] = acc
            return c
        lax.fori_loop(0, _NQ, l1b, 0)

        def round_body(r, c):
            def scan_q(q, carry):
                bv, bq = carry
                v = l1[pl.ds(q * 16, 16)]
                better = v > bv
                return (jnp.where(better, v, bv), jnp.where(better, q, bq))
            bv, bq = lax.fori_loop(0, _NQ, scan_q, (ninf, zero16))
            lv, lk = ninf, zero16
            for j in range(16):
                kidx = bq * 16 + j
                v = plsc.load_gather(mt, [kidx * 16 + lanes])
                better = v > lv
                lv = jnp.where(better, v, lv)
                lk = jnp.where(better, kidx, lk)
            handles = []
            bps = COL_BLK // 128
            for rr in range(16):
                b = lk[rr]
                off = ((b // bps) * P_CLU + rbase + rr) * COL_BLK \
                    + (b % bps) * 128
                handles.append(pltpu.async_copy(
                    sims_hbm.at[pl.ds(off, 128)],
                    blk.at[pl.ds(rr * 128, 128)], sem))
            for h in handles:
                h.wait()
            def prior_scan(rp, ws):
                mbk = pbk[pl.ds(rp * 16, 16)]
                ppos = pp[pl.ds(rp * 16, 16)]
                match = (mbk == lk) & (rp < r)
                bit = jnp.where(match,
                                jnp.left_shift(jnp.full((16,), 1, jnp.int32),
                                               ppos & 31), 0)
                wsel = ppos >> 5
                return tuple(w | jnp.where(wsel == i, bit, 0)
                             for i, w in enumerate(ws))
            ws = lax.fori_loop(0, TOP_K, prior_scan,
                               (zero16, zero16, zero16, zero16))
            m1, p1, m2 = ninf, zero16, ninf
            for w in range(4):
                def scan_j(j, carry, _w=w, _mask=ws[w]):
                    m1, p1, m2 = carry
                    x = plsc.load_gather(
                        blk, [lanes * 128 + (_w * 32 + j)])
                    excl = jnp.right_shift(_mask, j) & 1
                    x = jnp.where(excl == 1, -jnp.inf, x)
                    upd = x > m1
                    m2 = jnp.where(upd, m1, jnp.maximum(m2, x))
                    p1 = jnp.where(upd, _w * 32 + j, p1)
                    m1 = jnp.where(upd, x, m1)
                    return (m1, p1, m2)
                m1, p1, m2 = lax.fori_loop(0, 32, scan_j, (m1, p1, m2))
            pbk[pl.ds(r * 16, 16)] = lk
            pp[pl.ds(r * 16, 16)] = p1
            col = lk * 128 + p1
            pltpu.async_copy(lab_hbm.at[col], lab16, sem).wait()
            lv16 = lab16[...]
            labs[pl.ds(r * 16, 16)] = lv16
            plsc.store_scatter(outb, [lanes * TOP_K + r], lv16)
            plsc.store_scatter(mt, [lk * 16 + lanes], m2)
            acc = ninf
            for j in range(16):
                acc = jnp.maximum(
                    acc, plsc.load_gather(mt, [(bq * 16 + j) * 16 + lanes]))
            plsc.store_scatter(l1, [bq * 16 + lanes], acc)
            return c
        lax.fori_loop(0, TOP_K, round_body, 0)

        def majj(j, cnts):
            lvv = labs[pl.ds(j * 16, 16)]
            return tuple(cn + (lvv == cc).astype(jnp.int32)
                         for cc, cn in enumerate(cnts))
        cnts = lax.fori_loop(0, TOP_K, majj,
                             tuple(zero16 for _ in range(NUM_CLASSES)))
        bestc, bestn = zero16, cnts[0]
        for cc in range(1, NUM_CLASSES):
            better = cnts[cc] > bestn
            bestn = jnp.where(better, cnts[cc], bestn)
            bestc = jnp.where(better, jnp.full((16,), cc, jnp.int32), bestc)
        majb[...] = bestc
        pltpu.sync_copy(majb, majout_hbm.at[pl.ds(rbase, 16)])
        pltpu.sync_copy(outb, labout_hbm.at[pl.ds(rbase * TOP_K, 16 * TOP_K)])
        return _
    lax.fori_loop(0, _NG, group_body, 0)


def _sc_topk(sims2, m_t, labels):
    mesh = plsc.VectorSubcoreMesh(core_axis_name="c", subcore_axis_name="s")
    f = functools.partial(
        pl.kernel,
        out_type=[jax.ShapeDtypeStruct((P_CLU * TOP_K,), jnp.int32),
                  jax.ShapeDtypeStruct((P_CLU,), jnp.int32)],
        mesh=mesh,
        compiler_params=pltpu.CompilerParams(use_tc_tiling_on_sc=False,
                                             needs_layout_passes=False),
        scratch_types=[
            pltpu.VMEM((_RPW * NB,), jnp.float32),
            pltpu.VMEM((NB * 16,), jnp.float32),
            pltpu.VMEM((_NQ * 16,), jnp.float32),
            pltpu.VMEM((16 * 128,), jnp.float32),
            pltpu.VMEM((16 * TOP_K,), jnp.int32),
            pltpu.VMEM((16,), jnp.int32),
            pltpu.VMEM((TOP_K * 16,), jnp.int32),
            pltpu.VMEM((16,), jnp.int32),
            pltpu.VMEM((TOP_K * 16,), jnp.int32),
            pltpu.VMEM((TOP_K * 16,), jnp.int32),
            pltpu.SemaphoreType.DMA,
        ])(_sc2_body)
    lab_flat, maj = f(sims2, m_t.reshape(-1), labels)
    return lab_flat, maj


_PPW = N_PIX // _NW
_CHUNK = 128


def _sc1_body(emb_hbm, ci_hbm, zeros_hbm, out_hbm, rows_v, idxv, acc_sh, sem):
    c = lax.axis_index("c")
    s = lax.axis_index("s")
    wid = s * _NC + c
    pltpu.sync_copy(zeros_hbm, acc_sh.at[pl.ds(s * 128, 128)])
    plsc.subcore_barrier()
    def win(t, _):
        base = wid * _PPW + t * _CHUNK
        pltpu.sync_copy(ci_hbm.at[pl.ds(base, _CHUNK)], idxv)
        pltpu.sync_copy(emb_hbm.at[pl.ds(base, _CHUNK)], rows_v)
        pltpu.sync_copy(rows_v, acc_sh.at[idxv], add=True)
        return _
    lax.fori_loop(0, _PPW // _CHUNK, win, 0)
    plsc.subcore_barrier()
    pltpu.sync_copy(acc_sh.at[pl.ds(s * 128, 128)],
                    out_hbm.at[pl.ds(c * P_CLU + s * 128, 128)])


def _sc_scatter_add(emb, ci, zeros):
    mesh = plsc.VectorSubcoreMesh(core_axis_name="c", subcore_axis_name="s")
    f = functools.partial(
        pl.kernel,
        out_type=jax.ShapeDtypeStruct((_NC * P_CLU, D), jnp.float32),
        mesh=mesh,
        compiler_params=pltpu.CompilerParams(use_tc_tiling_on_sc=False,
                                             needs_layout_passes=False),
        scratch_types=[
            pltpu.VMEM((_CHUNK, D), jnp.float32),
            pltpu.VMEM((_CHUNK,), jnp.int32),
            pltpu.VMEM_SHARED((P_CLU, D), jnp.float32),
            pltpu.SemaphoreType.DMA,
        ])(_sc1_body)
    return f(emb, ci, zeros)


def _sc3_body(maj_hbm, lab_hbm, ci_hbm, pred_hbm, topk_hbm,
              majv, labv, cidx, predb, outb, sem):
    wid = lax.axis_index("s") * _NC + lax.axis_index("c")
    p0 = wid * _PPW
    pltpu.sync_copy(maj_hbm, majv)
    pltpu.sync_copy(lab_hbm, labv)
    lanes = lax.iota(jnp.int32, 16)

    def chunk_body(t, _):
        base = p0 + t * _CHUNK
        pltpu.sync_copy(ci_hbm.at[pl.ds(base, _CHUNK)], cidx)
        def sub_body(u, __):
            c16 = cidx[pl.ds(u * 16, 16)]
            pred16 = plsc.load_gather(majv, [c16])
            predb[pl.ds(u * 16, 16)] = pred16
            lpix = u * 16 + lanes
            for j in range(TOP_K):
                l16 = plsc.load_gather(labv, [c16 * TOP_K + j])
                plsc.store_scatter(outb, [lpix * TOP_K + j], l16)
            return __
        lax.fori_loop(0, _CHUNK // 16, sub_body, 0)
        pltpu.sync_copy(predb, pred_hbm.at[pl.ds(base, _CHUNK)])
        pltpu.sync_copy(outb, topk_hbm.at[pl.ds(base * TOP_K,
                                                _CHUNK * TOP_K)])
        return _
    lax.fori_loop(0, _PPW // _CHUNK, chunk_body, 0)


def _sc_broadcast(maj, lab_flat, ci):
    mesh = plsc.VectorSubcoreMesh(core_axis_name="c", subcore_axis_name="s")
    f = functools.partial(
        pl.kernel,
        out_type=[jax.ShapeDtypeStruct((N_PIX,), jnp.int32),
                  jax.ShapeDtypeStruct((N_PIX * TOP_K,), jnp.int32)],
        mesh=mesh,
        compiler_params=pltpu.CompilerParams(use_tc_tiling_on_sc=False,
                                             needs_layout_passes=False),
        scratch_types=[
            pltpu.VMEM((P_CLU,), jnp.int32),
            pltpu.VMEM((P_CLU * TOP_K,), jnp.int32),
            pltpu.VMEM((_CHUNK,), jnp.int32),
            pltpu.VMEM((_CHUNK,), jnp.int32),
            pltpu.VMEM((_CHUNK * TOP_K,), jnp.int32),
            pltpu.SemaphoreType.DMA,
        ])(_sc3_body)
    pred, topk_flat = f(maj, lab_flat, ci)
    return pred, topk_flat.reshape(N_PIX, TOP_K)


def kernel(cluster_embedding, cluster_index, memory_prototype,
           memory_prototype_label):
    ci = cluster_index.astype(jnp.int32)

    zeros = jnp.zeros((128, D), jnp.float32)
    acc = _sc_scatter_add(cluster_embedding, ci, zeros).reshape(2, P_CLU, D)

    mem_pad = jnp.pad(memory_prototype, ((0, K_PAD - K_MEM), (0, 0)))
    sims, blk_max = _tc_sims(acc, mem_pad)
    blk_max = jnp.transpose(blk_max, (1, 0, 2)).reshape(P_CLU, NB)

    sims2 = sims.reshape(P_CLU * K_PAD)
    lab_flat, maj = _sc_topk(sims2, blk_max,
                             memory_prototype_label.astype(jnp.int32))

    semantic_pred, semantic_topk = _sc_broadcast(maj, lab_flat, ci)
    return semantic_pred, semantic_topk


# speedup vs baseline: 1.6206x; 1.2976x over previous
import functools

import jax
import jax.numpy as jnp
from jax import lax
from jax.experimental import pallas as pl
from jax.experimental.pallas import tpu as pltpu
from jax.experimental.pallas import tpu_sc as plsc

N_PIX = 65536
D = 128
K_MEM = 100000
P_CLU = 2048
NUM_CLASSES = 21
TOP_K = 20

NB = 784
K_PAD = NB * 128
COL_BLK = 2048
NEG = -3.4e38


def _tc_sims_body(acc_ref, mem_ref, sims_ref, m_ref, ph_ref):
    i = pl.program_id(0)

    @pl.when(i == 0)
    def _():
        proto = acc_ref[0] + acc_ref[1]
        pn = jnp.sqrt(jnp.sum(proto * proto, axis=1, keepdims=True))
        ph_ref[...] = (proto / (pn + 1e-12)).astype(jnp.bfloat16)

    mb = mem_ref[...]
    mn = jnp.sqrt(jnp.sum(mb * mb, axis=1, keepdims=True))
    mh = (mb / (mn + 1e-12)).astype(jnp.bfloat16)
    s = lax.dot_general(ph_ref[...], mh, (((1,), (1,)), ((), ())),
                        preferred_element_type=jnp.float32)

    @pl.when(i != K_MEM // COL_BLK)
    def _():
        sims_ref[0] = s.reshape(P_CLU, COL_BLK // 128, 128)
        m_ref[0] = jnp.max(s.reshape(P_CLU, COL_BLK // 128, 128), axis=2)

    @pl.when(i == K_MEM // COL_BLK)
    def _():
        col = i * COL_BLK + lax.broadcasted_iota(jnp.int32, s.shape, 1)
        sm = jnp.where(col < K_MEM, s, NEG)
        sims_ref[0] = sm.reshape(P_CLU, COL_BLK // 128, 128)
        m_ref[0] = jnp.max(sm.reshape(P_CLU, COL_BLK // 128, 128), axis=2)


def _tc_sims(acc, mem_pad):
    grid = (K_PAD // COL_BLK,)
    return pl.pallas_call(
        _tc_sims_body,
        grid=grid,
        in_specs=[
            pl.BlockSpec((2, P_CLU, D), lambda i: (0, 0, 0)),
            pl.BlockSpec((COL_BLK, D), lambda i: (i, 0)),
        ],
        out_specs=[
            pl.BlockSpec((1, P_CLU, COL_BLK // 128, 128), lambda i: (i, 0, 0, 0)),
            pl.BlockSpec((1, P_CLU, COL_BLK // 128), lambda i: (i, 0, 0)),
        ],
        out_shape=[
            jax.ShapeDtypeStruct((K_PAD // COL_BLK, P_CLU, COL_BLK // 128, 128),
                                 jnp.float32),
            jax.ShapeDtypeStruct((K_PAD // COL_BLK, P_CLU, COL_BLK // 128),
                                 jnp.float32),
        ],
        scratch_shapes=[pltpu.VMEM((P_CLU, D), jnp.bfloat16)],
    )(acc, mem_pad)


_NC = 2
_NW = 32
_RPW = P_CLU // _NW
_NG = _RPW // 16
_NQ = NB // 16


def _sc2_body(sims_hbm, m_hbm, lab_hbm, labout_hbm, majout_hbm,
              m_slab, mt, l1, blk, outb, majb, labs, lab16, pbk, pp, sem):
    wid = lax.axis_index("s") * _NC + lax.axis_index("c")
    r0 = wid * _RPW
    pltpu.sync_copy(m_hbm.at[pl.ds(r0 * NB, _RPW * NB)], m_slab)
    lanes = lax.iota(jnp.int32, 16)
    ninf = jnp.full((16,), -jnp.inf, jnp.float32)
    zero16 = jnp.zeros((16,), jnp.int32)

    def group_body(g, _):
        rbase = r0 + g * 16
        def mtb(k, c):
            v = plsc.load_gather(
                m_slab, [(g * 16 + lanes) * NB + k])
            mt[pl.ds(k * 16, 16)] = v
            return c
        lax.fori_loop(0, NB, mtb, 0)

        def l1b(q, c):
            acc = ninf
            for j in range(16):
                acc = jnp.maximum(acc, mt[pl.ds((q * 16 + j) * 16, 16)])
            l1[pl.ds(q * 16, 16)] = acc
            return c
        lax.fori_loop(0, _NQ, l1b, 0)

        def round_body(r, c):
            def scan_q(q, carry):
                bv, bq = carry
                v = l1[pl.ds(q * 16, 16)]
                better = v > bv
                return (jnp.where(better, v, bv), jnp.where(better, q, bq))
            bv, bq = lax.fori_loop(0, _NQ, scan_q, (ninf, zero16))
            lv, lk = ninf, zero16
            for j in range(16):
                kidx = bq * 16 + j
                v = plsc.load_gather(mt, [kidx * 16 + lanes])
                better = v > lv
                lv = jnp.where(better, v, lv)
                lk = jnp.where(better, kidx, lk)
            handles = []
            bps = COL_BLK // 128
            for rr in range(16):
                b = lk[rr]
                off = ((b // bps) * P_CLU + rbase + rr) * COL_BLK \
                    + (b % bps) * 128
                handles.append(pltpu.async_copy(
                    sims_hbm.at[pl.ds(off, 128)],
                    blk.at[pl.ds(rr * 128, 128)], sem))
            for h in handles:
                h.wait()
            def prior_scan(rp, ws):
                mbk = pbk[pl.ds(rp * 16, 16)]
                ppos = pp[pl.ds(rp * 16, 16)]
                match = (mbk == lk) & (rp < r)
                bit = jnp.where(match,
                                jnp.left_shift(jnp.full((16,), 1, jnp.int32),
                                               ppos & 31), 0)
                wsel = ppos >> 5
                return tuple(w | jnp.where(wsel == i, bit, 0)
                             for i, w in enumerate(ws))
            ws = lax.fori_loop(0, TOP_K, prior_scan,
                               (zero16, zero16, zero16, zero16))
            m1, p1, m2 = ninf, zero16, ninf
            for w in range(4):
                def scan_j(j, carry, _w=w, _mask=ws[w]):
                    m1, p1, m2 = carry
                    x = plsc.load_gather(
                        blk, [lanes * 128 + (_w * 32 + j)])
                    excl = jnp.right_shift(_mask, j) & 1
                    x = jnp.where(excl == 1, -jnp.inf, x)
                    upd = x > m1
                    m2 = jnp.where(upd, m1, jnp.maximum(m2, x))
                    p1 = jnp.where(upd, _w * 32 + j, p1)
                    m1 = jnp.where(upd, x, m1)
                    return (m1, p1, m2)
                m1, p1, m2 = lax.fori_loop(0, 32, scan_j, (m1, p1, m2))
            pbk[pl.ds(r * 16, 16)] = lk
            pp[pl.ds(r * 16, 16)] = p1
            col = lk * 128 + p1
            pltpu.async_copy(lab_hbm.at[col], lab16, sem).wait()
            lv16 = lab16[...]
            labs[pl.ds(r * 16, 16)] = lv16
            plsc.store_scatter(outb, [lanes * TOP_K + r], lv16)
            plsc.store_scatter(mt, [lk * 16 + lanes], m2)
            acc = ninf
            for j in range(16):
                acc = jnp.maximum(
                    acc, plsc.load_gather(mt, [(bq * 16 + j) * 16 + lanes]))
            plsc.store_scatter(l1, [bq * 16 + lanes], acc)
            return c
        lax.fori_loop(0, TOP_K, round_body, 0)

        def majj(j, cnts):
            lvv = labs[pl.ds(j * 16, 16)]
            return tuple(cn + (lvv == cc).astype(jnp.int32)
                         for cc, cn in enumerate(cnts))
        cnts = lax.fori_loop(0, TOP_K, majj,
                             tuple(zero16 for _ in range(NUM_CLASSES)))
        bestc, bestn = zero16, cnts[0]
        for cc in range(1, NUM_CLASSES):
            better = cnts[cc] > bestn
            bestn = jnp.where(better, cnts[cc], bestn)
            bestc = jnp.where(better, jnp.full((16,), cc, jnp.int32), bestc)
        majb[...] = bestc
        pltpu.sync_copy(majb, majout_hbm.at[pl.ds(rbase, 16)])
        pltpu.sync_copy(outb, labout_hbm.at[pl.ds(rbase * TOP_K, 16 * TOP_K)])
        return _
    lax.fori_loop(0, _NG, group_body, 0)


def _sc_topk(sims2, m_t, labels):
    mesh = plsc.VectorSubcoreMesh(core_axis_name="c", subcore_axis_name="s")
    f = functools.partial(
        pl.kernel,
        out_type=[jax.ShapeDtypeStruct((P_CLU * TOP_K,), jnp.int32),
                  jax.ShapeDtypeStruct((P_CLU,), jnp.int32)],
        mesh=mesh,
        compiler_params=pltpu.CompilerParams(use_tc_tiling_on_sc=False,
                                             needs_layout_passes=False),
        scratch_types=[
            pltpu.VMEM((_RPW * NB,), jnp.float32),
            pltpu.VMEM((NB * 16,), jnp.float32),
            pltpu.VMEM((_NQ * 16,), jnp.float32),
            pltpu.VMEM((16 * 128,), jnp.float32),
            pltpu.VMEM((16 * TOP_K,), jnp.int32),
            pltpu.VMEM((16,), jnp.int32),
            pltpu.VMEM((TOP_K * 16,), jnp.int32),
            pltpu.VMEM((16,), jnp.int32),
            pltpu.VMEM((TOP_K * 16,), jnp.int32),
            pltpu.VMEM((TOP_K * 16,), jnp.int32),
            pltpu.SemaphoreType.DMA,
        ])(_sc2_body)
    lab_flat, maj = f(sims2, m_t.reshape(-1), labels)
    return lab_flat, maj


_PPW = N_PIX // _NW
_CHUNK = 128


def _sc1_body(emb_hbm, ci_hbm, zeros_hbm, out_hbm, rows_v, idxv, acc_sh, sem):
    c = lax.axis_index("c")
    s = lax.axis_index("s")
    wid = s * _NC + c
    pltpu.sync_copy(zeros_hbm, acc_sh.at[pl.ds(s * 128, 128)])
    plsc.subcore_barrier()
    def win(t, _):
        base = wid * _PPW + t * _CHUNK
        pltpu.sync_copy(ci_hbm.at[pl.ds(base, _CHUNK)], idxv)
        pltpu.sync_copy(emb_hbm.at[pl.ds(base, _CHUNK)], rows_v)
        pltpu.sync_copy(rows_v, acc_sh.at[idxv], add=True)
        return _
    lax.fori_loop(0, _PPW // _CHUNK, win, 0)
    plsc.subcore_barrier()
    pltpu.sync_copy(acc_sh.at[pl.ds(s * 128, 128)],
                    out_hbm.at[pl.ds(c * P_CLU + s * 128, 128)])


def _sc_scatter_add(emb, ci, zeros):
    mesh = plsc.VectorSubcoreMesh(core_axis_name="c", subcore_axis_name="s")
    f = functools.partial(
        pl.kernel,
        out_type=jax.ShapeDtypeStruct((_NC * P_CLU, D), jnp.float32),
        mesh=mesh,
        compiler_params=pltpu.CompilerParams(use_tc_tiling_on_sc=False,
                                             needs_layout_passes=False),
        scratch_types=[
            pltpu.VMEM((_CHUNK, D), jnp.float32),
            pltpu.VMEM((_CHUNK,), jnp.int32),
            pltpu.VMEM_SHARED((P_CLU, D), jnp.float32),
            pltpu.SemaphoreType.DMA,
        ])(_sc1_body)
    return f(emb, ci, zeros)


def _sc3_body(maj_hbm, lab_hbm, ci_hbm, pred_hbm, topk_hbm,
              majv, labv, cidx, predb, outb, sem):
    wid = lax.axis_index("s") * _NC + lax.axis_index("c")
    p0 = wid * _PPW
    pltpu.sync_copy(maj_hbm, majv)
    pltpu.sync_copy(lab_hbm, labv)
    lanes = lax.iota(jnp.int32, 16)

    def chunk_body(t, _):
        base = p0 + t * _CHUNK
        pltpu.sync_copy(ci_hbm.at[pl.ds(base, _CHUNK)], cidx)
        def sub_body(u, __):
            c16 = cidx[pl.ds(u * 16, 16)]
            pred16 = plsc.load_gather(majv, [c16])
            predb[pl.ds(u * 16, 16)] = pred16
            lpix = u * 16 + lanes
            for j in range(TOP_K):
                l16 = plsc.load_gather(labv, [c16 * TOP_K + j])
                plsc.store_scatter(outb, [lpix * TOP_K + j], l16)
            return __
        lax.fori_loop(0, _CHUNK // 16, sub_body, 0)
        pltpu.sync_copy(predb, pred_hbm.at[pl.ds(base, _CHUNK)])
        pltpu.sync_copy(outb, topk_hbm.at[pl.ds(base * TOP_K,
                                                _CHUNK * TOP_K)])
        return _
    lax.fori_loop(0, _PPW // _CHUNK, chunk_body, 0)


def _sc_broadcast(maj, lab_flat, ci):
    mesh = plsc.VectorSubcoreMesh(core_axis_name="c", subcore_axis_name="s")
    f = functools.partial(
        pl.kernel,
        out_type=[jax.ShapeDtypeStruct((N_PIX,), jnp.int32),
                  jax.ShapeDtypeStruct((N_PIX * TOP_K,), jnp.int32)],
        mesh=mesh,
        compiler_params=pltpu.CompilerParams(use_tc_tiling_on_sc=False,
                                             needs_layout_passes=False),
        scratch_types=[
            pltpu.VMEM((P_CLU,), jnp.int32),
            pltpu.VMEM((P_CLU * TOP_K,), jnp.int32),
            pltpu.VMEM((_CHUNK,), jnp.int32),
            pltpu.VMEM((_CHUNK,), jnp.int32),
            pltpu.VMEM((_CHUNK * TOP_K,), jnp.int32),
            pltpu.SemaphoreType.DMA,
        ])(_sc3_body)
    pred, topk_flat = f(maj, lab_flat, ci)
    return pred, topk_flat.reshape(N_PIX, TOP_K)


def kernel(cluster_embedding, cluster_index, memory_prototype,
           memory_prototype_label):
    ci = cluster_index.astype(jnp.int32)

    zeros = jnp.zeros((128, D), jnp.float32)
    acc = _sc_scatter_add(cluster_embedding, ci, zeros).reshape(2, P_CLU, D)

    mem_pad = jnp.pad(memory_prototype, ((0, K_PAD - K_MEM), (0, 0)))
    sims, blk_max = _tc_sims(acc, mem_pad)
    blk_max = jnp.transpose(blk_max, (1, 0, 2)).reshape(P_CLU, NB)

    sims2 = sims.reshape(P_CLU * K_PAD)
    lab_flat, maj = _sc_topk(sims2, blk_max,
                             memory_prototype_label.astype(jnp.int32))

    semantic_pred, semantic_topk = _sc_broadcast(maj, lab_flat, ci)
    return semantic_pred, semantic_topk


# speedup vs baseline: 1.6223x; 1.0010x over previous
import functools

import jax
import jax.numpy as jnp
from jax import lax
from jax.experimental import pallas as pl
from jax.experimental.pallas import tpu as pltpu
from jax.experimental.pallas import tpu_sc as plsc

N_PIX = 65536
D = 128
K_MEM = 100000
P_CLU = 2048
NUM_CLASSES = 21
TOP_K = 20

COL_BLK = 2048
K_PAD = ((K_MEM + COL_BLK - 1) // COL_BLK) * COL_BLK
NB = K_PAD // 128
NEG = -3.4e38


def _tc_sims_body(acc_ref, mem_ref, sims_ref, m_ref, ph_ref):
    i = pl.program_id(0)

    @pl.when(i == 0)
    def _():
        proto = acc_ref[0] + acc_ref[1]
        pn = jnp.sqrt(jnp.sum(proto * proto, axis=1, keepdims=True))
        ph_ref[...] = (proto / (pn + 1e-12)).astype(jnp.bfloat16)

    mb = mem_ref[...]
    mn = jnp.sqrt(jnp.sum(mb * mb, axis=1, keepdims=True))
    mh = (mb / (mn + 1e-12)).astype(jnp.bfloat16)
    s = lax.dot_general(ph_ref[...], mh, (((1,), (1,)), ((), ())),
                        preferred_element_type=jnp.float32)

    @pl.when(i != K_MEM // COL_BLK)
    def _():
        sims_ref[0] = s.reshape(P_CLU, COL_BLK // 128, 128)
        m_ref[0] = jnp.max(s.reshape(P_CLU, COL_BLK // 128, 128), axis=2)

    @pl.when(i == K_MEM // COL_BLK)
    def _():
        col = i * COL_BLK + lax.broadcasted_iota(jnp.int32, s.shape, 1)
        sm = jnp.where(col < K_MEM, s, NEG)
        sims_ref[0] = sm.reshape(P_CLU, COL_BLK // 128, 128)
        m_ref[0] = jnp.max(sm.reshape(P_CLU, COL_BLK // 128, 128), axis=2)


def _tc_sims(acc, mem_pad):
    grid = (K_PAD // COL_BLK,)
    return pl.pallas_call(
        _tc_sims_body,
        grid=grid,
        in_specs=[
            pl.BlockSpec((2, P_CLU, D), lambda i: (0, 0, 0)),
            pl.BlockSpec((COL_BLK, D), lambda i: (i, 0)),
        ],
        out_specs=[
            pl.BlockSpec((1, P_CLU, COL_BLK // 128, 128), lambda i: (i, 0, 0, 0)),
            pl.BlockSpec((1, P_CLU, COL_BLK // 128), lambda i: (i, 0, 0)),
        ],
        out_shape=[
            jax.ShapeDtypeStruct((K_PAD // COL_BLK, P_CLU, COL_BLK // 128, 128),
                                 jnp.float32),
            jax.ShapeDtypeStruct((K_PAD // COL_BLK, P_CLU, COL_BLK // 128),
                                 jnp.float32),
        ],
        scratch_shapes=[pltpu.VMEM((P_CLU, D), jnp.bfloat16)],
    )(acc, mem_pad)


_NC = 2
_NW = 32
_RPW = P_CLU // _NW
_NG = _RPW // 16
_NQ = NB // 16


def _sc2_body(sims_hbm, m_hbm, lab_hbm, labout_hbm, majout_hbm,
              m_slab, mt, l1, blk, outb, majb, labs, lab16, pbk, pp, sem):
    wid = lax.axis_index("s") * _NC + lax.axis_index("c")
    r0 = wid * _RPW
    pltpu.sync_copy(m_hbm.at[pl.ds(r0 * NB, _RPW * NB)], m_slab)
    lanes = lax.iota(jnp.int32, 16)
    ninf = jnp.full((16,), -jnp.inf, jnp.float32)
    zero16 = jnp.zeros((16,), jnp.int32)

    def group_body(g, _):
        rbase = r0 + g * 16
        def mtb(k, c):
            v = plsc.load_gather(
                m_slab, [(g * 16 + lanes) * NB + k])
            mt[pl.ds(k * 16, 16)] = v
            return c
        lax.fori_loop(0, NB, mtb, 0)

        def l1b(q, c):
            acc = ninf
            for j in range(16):
                acc = jnp.maximum(acc, mt[pl.ds((q * 16 + j) * 16, 16)])
            l1[pl.ds(q * 16, 16)] = acc
            return c
        lax.fori_loop(0, _NQ, l1b, 0)

        def round_body(r, c):
            def scan_q(q, carry):
                bv, bq = carry
                v = l1[pl.ds(q * 16, 16)]
                better = v > bv
                return (jnp.where(better, v, bv), jnp.where(better, q, bq))
            bv, bq = lax.fori_loop(0, _NQ, scan_q, (ninf, zero16))
            lv, lk = ninf, zero16
            for j in range(16):
                kidx = bq * 16 + j
                v = plsc.load_gather(mt, [kidx * 16 + lanes])
                better = v > lv
                lv = jnp.where(better, v, lv)
                lk = jnp.where(better, kidx, lk)
            handles = []
            bps = COL_BLK // 128
            for rr in range(16):
                b = lk[rr]
                off = ((b // bps) * P_CLU + rbase + rr) * COL_BLK \
                    + (b % bps) * 128
                handles.append(pltpu.async_copy(
                    sims_hbm.at[pl.ds(off, 128)],
                    blk.at[pl.ds(rr * 128, 128)], sem))
            for h in handles:
                h.wait()
            def prior_scan(rp, ws):
                mbk = pbk[pl.ds(rp * 16, 16)]
                ppos = pp[pl.ds(rp * 16, 16)]
                match = (mbk == lk) & (rp < r)
                bit = jnp.where(match,
                                jnp.left_shift(jnp.full((16,), 1, jnp.int32),
                                               ppos & 31), 0)
                wsel = ppos >> 5
                return tuple(w | jnp.where(wsel == i, bit, 0)
                             for i, w in enumerate(ws))
            ws = lax.fori_loop(0, TOP_K, prior_scan,
                               (zero16, zero16, zero16, zero16))
            m1, p1, m2 = ninf, zero16, ninf
            for w in range(4):
                def scan_j(j, carry, _w=w, _mask=ws[w]):
                    m1, p1, m2 = carry
                    x = plsc.load_gather(
                        blk, [lanes * 128 + (_w * 32 + j)])
                    excl = jnp.right_shift(_mask, j) & 1
                    x = jnp.where(excl == 1, -jnp.inf, x)
                    upd = x > m1
                    m2 = jnp.where(upd, m1, jnp.maximum(m2, x))
                    p1 = jnp.where(upd, _w * 32 + j, p1)
                    m1 = jnp.where(upd, x, m1)
                    return (m1, p1, m2)
                m1, p1, m2 = lax.fori_loop(0, 32, scan_j, (m1, p1, m2))
            pbk[pl.ds(r * 16, 16)] = lk
            pp[pl.ds(r * 16, 16)] = p1
            col = lk * 128 + p1
            pltpu.async_copy(lab_hbm.at[col], lab16, sem).wait()
            lv16 = lab16[...]
            labs[pl.ds(r * 16, 16)] = lv16
            plsc.store_scatter(outb, [lanes * TOP_K + r], lv16)
            plsc.store_scatter(mt, [lk * 16 + lanes], m2)
            acc = ninf
            for j in range(16):
                acc = jnp.maximum(
                    acc, plsc.load_gather(mt, [(bq * 16 + j) * 16 + lanes]))
            plsc.store_scatter(l1, [bq * 16 + lanes], acc)
            return c
        lax.fori_loop(0, TOP_K, round_body, 0)

        def majj(j, cnts):
            lvv = labs[pl.ds(j * 16, 16)]
            return tuple(cn + (lvv == cc).astype(jnp.int32)
                         for cc, cn in enumerate(cnts))
        cnts = lax.fori_loop(0, TOP_K, majj,
                             tuple(zero16 for _ in range(NUM_CLASSES)))
        bestc, bestn = zero16, cnts[0]
        for cc in range(1, NUM_CLASSES):
            better = cnts[cc] > bestn
            bestn = jnp.where(better, cnts[cc], bestn)
            bestc = jnp.where(better, jnp.full((16,), cc, jnp.int32), bestc)
        majb[...] = bestc
        pltpu.sync_copy(majb, majout_hbm.at[pl.ds(rbase, 16)])
        pltpu.sync_copy(outb, labout_hbm.at[pl.ds(rbase * TOP_K, 16 * TOP_K)])
        return _
    lax.fori_loop(0, _NG, group_body, 0)


def _sc_topk(sims2, m_t, labels):
    mesh = plsc.VectorSubcoreMesh(core_axis_name="c", subcore_axis_name="s")
    f = functools.partial(
        pl.kernel,
        out_type=[jax.ShapeDtypeStruct((P_CLU * TOP_K,), jnp.int32),
                  jax.ShapeDtypeStruct((P_CLU,), jnp.int32)],
        mesh=mesh,
        compiler_params=pltpu.CompilerParams(use_tc_tiling_on_sc=False,
                                             needs_layout_passes=False),
        scratch_types=[
            pltpu.VMEM((_RPW * NB,), jnp.float32),
            pltpu.VMEM((NB * 16,), jnp.float32),
            pltpu.VMEM((_NQ * 16,), jnp.float32),
            pltpu.VMEM((16 * 128,), jnp.float32),
            pltpu.VMEM((16 * TOP_K,), jnp.int32),
            pltpu.VMEM((16,), jnp.int32),
            pltpu.VMEM((TOP_K * 16,), jnp.int32),
            pltpu.VMEM((16,), jnp.int32),
            pltpu.VMEM((TOP_K * 16,), jnp.int32),
            pltpu.VMEM((TOP_K * 16,), jnp.int32),
            pltpu.SemaphoreType.DMA,
        ])(_sc2_body)
    lab_flat, maj = f(sims2, m_t.reshape(-1), labels)
    return lab_flat, maj


_PPW = N_PIX // _NW
_CHUNK = 128


def _sc1_body(emb_hbm, ci_hbm, zeros_hbm, out_hbm, rows_v, idxv, acc_sh, sem):
    c = lax.axis_index("c")
    s = lax.axis_index("s")
    wid = s * _NC + c
    pltpu.sync_copy(zeros_hbm, acc_sh.at[pl.ds(s * 128, 128)])
    plsc.subcore_barrier()
    def win(t, _):
        base = wid * _PPW + t * _CHUNK
        pltpu.sync_copy(ci_hbm.at[pl.ds(base, _CHUNK)], idxv)
        pltpu.sync_copy(emb_hbm.at[pl.ds(base, _CHUNK)], rows_v)
        pltpu.sync_copy(rows_v, acc_sh.at[idxv], add=True)
        return _
    lax.fori_loop(0, _PPW // _CHUNK, win, 0)
    plsc.subcore_barrier()
    pltpu.sync_copy(acc_sh.at[pl.ds(s * 128, 128)],
                    out_hbm.at[pl.ds(c * P_CLU + s * 128, 128)])


def _sc_scatter_add(emb, ci, zeros):
    mesh = plsc.VectorSubcoreMesh(core_axis_name="c", subcore_axis_name="s")
    f = functools.partial(
        pl.kernel,
        out_type=jax.ShapeDtypeStruct((_NC * P_CLU, D), jnp.float32),
        mesh=mesh,
        compiler_params=pltpu.CompilerParams(use_tc_tiling_on_sc=False,
                                             needs_layout_passes=False),
        scratch_types=[
            pltpu.VMEM((_CHUNK, D), jnp.float32),
            pltpu.VMEM((_CHUNK,), jnp.int32),
            pltpu.VMEM_SHARED((P_CLU, D), jnp.float32),
            pltpu.SemaphoreType.DMA,
        ])(_sc1_body)
    return f(emb, ci, zeros)


def _sc3_body(maj_hbm, lab_hbm, ci_hbm, pred_hbm, topk_hbm,
              majv, labv, cidx, predb, outb, sem):
    wid = lax.axis_index("s") * _NC + lax.axis_index("c")
    p0 = wid * _PPW
    pltpu.sync_copy(maj_hbm, majv)
    pltpu.sync_copy(lab_hbm, labv)
    lanes = lax.iota(jnp.int32, 16)

    def chunk_body(t, _):
        base = p0 + t * _CHUNK
        pltpu.sync_copy(ci_hbm.at[pl.ds(base, _CHUNK)], cidx)
        def sub_body(u, __):
            c16 = cidx[pl.ds(u * 16, 16)]
            pred16 = plsc.load_gather(majv, [c16])
            predb[pl.ds(u * 16, 16)] = pred16
            lpix = u * 16 + lanes
            for j in range(TOP_K):
                l16 = plsc.load_gather(labv, [c16 * TOP_K + j])
                plsc.store_scatter(outb, [lpix * TOP_K + j], l16)
            return __
        lax.fori_loop(0, _CHUNK // 16, sub_body, 0)
        pltpu.sync_copy(predb, pred_hbm.at[pl.ds(base, _CHUNK)])
        pltpu.sync_copy(outb, topk_hbm.at[pl.ds(base * TOP_K,
                                                _CHUNK * TOP_K)])
        return _
    lax.fori_loop(0, _PPW // _CHUNK, chunk_body, 0)


def _sc_broadcast(maj, lab_flat, ci):
    mesh = plsc.VectorSubcoreMesh(core_axis_name="c", subcore_axis_name="s")
    f = functools.partial(
        pl.kernel,
        out_type=[jax.ShapeDtypeStruct((N_PIX,), jnp.int32),
                  jax.ShapeDtypeStruct((N_PIX * TOP_K,), jnp.int32)],
        mesh=mesh,
        compiler_params=pltpu.CompilerParams(use_tc_tiling_on_sc=False,
                                             needs_layout_passes=False),
        scratch_types=[
            pltpu.VMEM((P_CLU,), jnp.int32),
            pltpu.VMEM((P_CLU * TOP_K,), jnp.int32),
            pltpu.VMEM((_CHUNK,), jnp.int32),
            pltpu.VMEM((_CHUNK,), jnp.int32),
            pltpu.VMEM((_CHUNK * TOP_K,), jnp.int32),
            pltpu.SemaphoreType.DMA,
        ])(_sc3_body)
    pred, topk_flat = f(maj, lab_flat, ci)
    return pred, topk_flat.reshape(N_PIX, TOP_K)


def kernel(cluster_embedding, cluster_index, memory_prototype,
           memory_prototype_label):
    ci = cluster_index.astype(jnp.int32)

    zeros = jnp.zeros((128, D), jnp.float32)
    acc = _sc_scatter_add(cluster_embedding, ci, zeros).reshape(2, P_CLU, D)

    mem_pad = jnp.pad(memory_prototype, ((0, K_PAD - K_MEM), (0, 0)))
    sims, blk_max = _tc_sims(acc, mem_pad)
    blk_max = jnp.transpose(blk_max, (1, 0, 2)).reshape(P_CLU, NB)

    sims2 = sims.reshape(P_CLU * K_PAD)
    lab_flat, maj = _sc_topk(sims2, blk_max,
                             memory_prototype_label.astype(jnp.int32))

    semantic_pred, semantic_topk = _sc_broadcast(maj, lab_flat, ci)
    return semantic_pred, semantic_topk
